# baseline (speedup 1.0000x reference)
import functools

import jax
import jax.numpy as jnp
from jax import lax
from jax.experimental import pallas as pl
from jax.experimental.pallas import tpu as pltpu
from jax.experimental.pallas import tpu_sc as plsc

_DIM = 2048
_NC, _NS = 2, 16
_NW = _NC * _NS
_C = 8
_D = 4
_P = 3


def kernel(position_ids, embed_table):
    b, s = position_ids.shape
    n = b * s
    b_per_w = n // _NW
    nch = b_per_w // _C
    idx = position_ids.reshape(n).astype(jnp.int32)
    mesh = plsc.VectorSubcoreMesh(core_axis_name="c", subcore_axis_name="s")

    @functools.partial(
        pl.kernel,
        out_type=jax.ShapeDtypeStruct((n, _DIM), jnp.float32),
        mesh=mesh,
        scratch_types=(
            [pltpu.VMEM((b_per_w,), jnp.int32)]
            + [pltpu.VMEM((_C, _DIM), jnp.float32)] * _D
            + [pltpu.SemaphoreType.DMA] * (2 * _D)
        ),
    )
    def gather_rows(table_hbm, idx_hbm, out_hbm, idx_v, *scratch):
        bufs = scratch[:_D]
        gsems = scratch[_D:2 * _D]
        wsems = scratch[2 * _D:]
        wid = lax.axis_index("c") * _NS + lax.axis_index("s")
        base = wid * b_per_w
        pltpu.sync_copy(idx_hbm.at[pl.ds(base, b_per_w)], idx_v)

        def start_gather(g, slot):
            pltpu.async_copy(
                table_hbm.at[idx_v.at[pl.ds(g * _C, _C)]], bufs[slot], gsems[slot])

        def wait_gather(slot):
            pltpu.make_async_copy(
                table_hbm.at[idx_v.at[pl.ds(0, _C)]], bufs[slot], gsems[slot]).wait()

        def start_write(g, slot):
            pltpu.async_copy(
                bufs[slot], out_hbm.at[pl.ds(base + g * _C, _C)], wsems[slot])

        def wait_write(slot):
            pltpu.make_async_copy(
                bufs[slot], out_hbm.at[pl.ds(base, _C)], wsems[slot]).wait()

        for j in range(_P):
            start_gather(j, j)

        @pl.loop(0, nch, step=_D)
        def _(g0):
            for k in range(_D):
                g = g0 + k
                slot = k
                pf_slot = (k + _P) % _D

                @pl.when(g + _P < nch)
                def _():
                    @pl.when(g + _P - _D >= 0)
                    def _():
                        wait_write(pf_slot)

                    start_gather(g + _P, pf_slot)

                wait_gather(slot)
                start_write(g, slot)

        for i in range(_D):
            wait_write((nch - _D + i) % _D)

    return gather_rows(embed_table, idx).reshape(b, s, _DIM)


# speedup vs baseline: 1.0142x; 1.0142x over previous
import functools

import jax
import jax.numpy as jnp
from jax import lax
from jax.experimental import pallas as pl
from jax.experimental.pallas import tpu as pltpu
from jax.experimental.pallas import tpu_sc as plsc
from jax._src.pallas import core as pallas_core
from jax._src.pallas import mpmd
from jax._src.pallas.mosaic import core as tpu_core

_DIM = 2048
_NC, _NS = 2, 16
_NW = _NC * _NS
_C = 8
_D = 4
_P = 3

_S = 4096
_R = 64
_IDXB = 128


def kernel(position_ids, embed_table):
    b, s = position_ids.shape
    n = b * s
    n_scs = _NC * _S
    n_tec = n - n_scs
    b_per_w = n_tec // _NW
    nch = b_per_w // _C
    idx = position_ids.reshape(n).astype(jnp.int32)

    scalar_mesh = plsc.ScalarSubcoreMesh(axis_name="c", num_cores=_NC)
    vector_mesh = plsc.VectorSubcoreMesh(core_axis_name="c", subcore_axis_name="s")
    v_vmem = pallas_core.CoreMemorySpace(tpu_core.MemorySpace.VMEM, vector_mesh)
    s_smem = pallas_core.CoreMemorySpace(tpu_core.MemorySpace.SMEM, scalar_mesh)

    def tec_fn(table_hbm, idx_hbm, out_hbm, idx_v, *scratch):
        bufs = scratch[:_D]
        gsems = scratch[_D:2 * _D]
        wsems = scratch[2 * _D:3 * _D]
        wid = lax.axis_index("c") * _NS + lax.axis_index("s")
        base = wid * b_per_w
        pltpu.sync_copy(idx_hbm.at[pl.ds(base, b_per_w)], idx_v)

        def start_gather(g, slot):
            pltpu.async_copy(
                table_hbm.at[idx_v.at[pl.ds(g * _C, _C)]], bufs[slot], gsems[slot])

        def wait_gather(slot):
            pltpu.make_async_copy(
                table_hbm.at[idx_v.at[pl.ds(0, _C)]], bufs[slot], gsems[slot]).wait()

        def start_write(g, slot):
            pltpu.async_copy(
                bufs[slot], out_hbm.at[pl.ds(base + g * _C, _C)], wsems[slot])

        def wait_write(slot):
            pltpu.make_async_copy(
                bufs[slot], out_hbm.at[pl.ds(base, _C)], wsems[slot]).wait()

        for j in range(_P):
            start_gather(j, j)

        @pl.loop(0, nch, step=_D)
        def _(g0):
            for k in range(_D):
                g = g0 + k
                slot = k
                pf_slot = (k + _P) % _D

                @pl.when(g + _P < nch)
                def _():
                    @pl.when(g + _P - _D >= 0)
                    def _():
                        wait_write(pf_slot)

                    start_gather(g + _P, pf_slot)

                wait_gather(slot)
                start_write(g, slot)

        for i in range(_D):
            wait_write((nch - _D + i) % _D)

    def scs_fn(table_hbm, idx_hbm, out_hbm, idx_v, *scratch):
        sc = scratch[3 * _D:]
        slabs = sc[0:2]
        idx_s = sc[2]
        isems = sc[3:5]
        gsems = sc[5:7]
        wsems = sc[7:9]
        cid = lax.axis_index("c")
        base = n_tec + cid * _S
        nu = _S // _IDXB
        nb = _S // _R

        def start_idx(u, k):
            pltpu.async_copy(
                idx_hbm.at[pl.ds(base + u * _IDXB, _IDXB)], idx_s.at[k], isems[k])

        def wait_idx(k):
            pltpu.make_async_copy(
                idx_hbm.at[pl.ds(base, _IDXB)], idx_s.at[k], isems[k]).wait()

        def start_batch(t, k, half, slot):
            @pl.loop(0, _R)
            def _(i):
                row = idx_s[k, half * _R + i]
                pltpu.async_copy(
                    table_hbm.at[pl.ds(row, 1)],
                    slabs[slot].at[cid, pl.ds(i, 1)],
                    gsems[slot])

        def wait_batch(slot):
            pltpu.make_async_copy(
                table_hbm.at[pl.ds(0, _R)], slabs[slot].at[cid], gsems[slot]).wait()

        def start_write(t, slot):
            pltpu.async_copy(
                slabs[slot].at[cid], out_hbm.at[pl.ds(base + t * _R, _R)],
                wsems[slot])

        def wait_write(slot):
            pltpu.make_async_copy(
                slabs[slot].at[cid], out_hbm.at[pl.ds(base, _R)],
                wsems[slot]).wait()

        start_idx(0, 0)
        start_idx(1, 1)

        @pl.loop(0, nu, step=2)
        def _(u0):
            for k in range(2):
                u = u0 + k
                wait_idx(k)
                for half in range(2):
                    t = 2 * u + half
                    slot = half

                    @pl.when(t >= 2)
                    def _():
                        wait_write(slot)

                    start_batch(t, k, half, slot)
                    wait_batch(slot)
                    start_write(t, slot)

                @pl.when(u + 2 < nu)
                def _():
                    start_idx(u + 2, k)

        wait_write(0)
        wait_write(1)

    scratch_types = (
        [v_vmem((b_per_w,), jnp.int32)]
        + [v_vmem((_C, _DIM), jnp.float32)] * _D
        + [pltpu.SemaphoreType.DMA @ vector_mesh] * (2 * _D)
        + [pltpu.VMEM_SHARED((_NC, _R, _DIM), jnp.float32)] * 2
        + [s_smem((2, _IDXB), jnp.int32)]
        + [pltpu.SemaphoreType.DMA @ scalar_mesh] * 6
    )

    run = mpmd.mpmd_map(
        [(vector_mesh, tec_fn), (scalar_mesh, scs_fn)],
        out_types=jax.ShapeDtypeStruct((n, _DIM), jnp.float32),
        scratch_types=scratch_types,
    )
    return run(embed_table, idx).reshape(b, s, _DIM)
